# baseline (device time: 153792 ns/iter reference)
import jax
import jax.numpy as jnp
from jax import lax
from jax.experimental import pallas as pl
from jax.experimental.pallas import tpu as pltpu

CHUNK = 32
NSLOT = 3


def kernel(x, W):
    logits = jnp.dot(x, W, preferred_element_type=jnp.float32)
    m_rows, n_half = logits.shape
    n_total = 2 * n_half
    n_blocks = m_rows // CHUNK
    half_rows = m_rows // 2
    n_fc = half_rows // CHUNK

    def body(logits_hbm, out_blk, recv_vmem, lbuf,
             dsend, drecv, fsend, frecv, lsem):
        i = pl.program_id(0)
        my_x = lax.axis_index("x")
        my_y = lax.axis_index("y")
        partner_y = (my_x, 1 - my_y)
        partner_x = (1 - my_x, my_y)

        def d_rdma(c):
            base = my_x * half_rows + c * CHUNK
            return pltpu.make_async_remote_copy(
                src_ref=logits_hbm.at[pl.ds(base, CHUNK)],
                dst_ref=recv_vmem.at[pl.ds(base, CHUNK)],
                send_sem=dsend.at[c],
                recv_sem=drecv.at[c],
                device_id=partner_y,
                device_id_type=pl.DeviceIdType.MESH,
            )

        def f_rdma(c):
            base = my_x * half_rows + c * CHUNK
            return pltpu.make_async_remote_copy(
                src_ref=recv_vmem.at[pl.ds(base, CHUNK)],
                dst_ref=recv_vmem.at[pl.ds(base, CHUNK)],
                send_sem=fsend.at[c],
                recv_sem=frecv.at[c],
                device_id=partner_x,
                device_id_type=pl.DeviceIdType.MESH,
            )

        def lcopy(blk, slot):
            return pltpu.make_async_copy(
                logits_hbm.at[pl.ds(blk * CHUNK, CHUNK)],
                lbuf.at[slot],
                lsem.at[slot],
            )

        @pl.when(i == 0)
        def _():
            barrier = pltpu.get_barrier_semaphore()
            for nbr in (partner_y, partner_x):
                pl.semaphore_signal(
                    barrier, inc=1, device_id=nbr,
                    device_id_type=pl.DeviceIdType.MESH,
                )
            pl.semaphore_wait(barrier, 2)
            for c in range(n_fc):
                d_rdma(c).start()
            for b in range(NSLOT):
                lcopy(b, b).start()

        @pl.when(jnp.logical_and(i >= 1, i + NSLOT - 1 < n_blocks))
        def _():
            lcopy(i + NSLOT - 1, (i + NSLOT - 1) % NSLOT).start()

        @pl.when(i < n_fc)
        def _():
            d_rdma(i).wait_recv()
            f_rdma(i).start()

        is_forwarded = (i // n_fc) != my_x
        @pl.when(is_forwarded)
        def _():
            f_rdma(i % n_fc).wait_recv()

        @pl.when(i >= n_fc)
        def _():
            d_rdma(i - n_fc).wait_send()
            f_rdma(i - n_fc).wait_send()

        lcopy(i, i % NSLOT).wait()
        v_l = lbuf[i % NSLOT]
        v_r = recv_vmem[pl.ds(i * CHUNK, CHUNK), :]
        m = jnp.maximum(
            jnp.max(v_l, axis=-1, keepdims=True),
            jnp.max(v_r, axis=-1, keepdims=True),
        )
        e_l = jnp.exp(v_l - m)
        e_r = jnp.exp(v_r - m)
        r = 1.0 / (
            jnp.sum(e_l, axis=-1, keepdims=True)
            + jnp.sum(e_r, axis=-1, keepdims=True)
        )
        out_blk[:, pl.ds(my_y * n_half, n_half)] = e_l * r
        out_blk[:, pl.ds((1 - my_y) * n_half, n_half)] = e_r * r

    return pl.pallas_call(
        body,
        grid=(n_blocks,),
        out_shape=jax.ShapeDtypeStruct((m_rows, n_total), jnp.float32),
        in_specs=[
            pl.BlockSpec(memory_space=pl.ANY),
        ],
        out_specs=pl.BlockSpec((CHUNK, n_total), lambda i: (i, 0)),
        scratch_shapes=[
            pltpu.VMEM((m_rows, n_half), jnp.float32),
            pltpu.VMEM((NSLOT, CHUNK, n_half), jnp.float32),
            pltpu.SemaphoreType.DMA((n_fc,)),
            pltpu.SemaphoreType.DMA((n_fc,)),
            pltpu.SemaphoreType.DMA((n_fc,)),
            pltpu.SemaphoreType.DMA((n_fc,)),
            pltpu.SemaphoreType.DMA((NSLOT,)),
        ],
        compiler_params=pltpu.CompilerParams(
            collective_id=0,
            dimension_semantics=("arbitrary",),
        ),
    )(logits)


# device time: 150351 ns/iter; 1.0229x vs baseline; 1.0229x over previous
import jax
import jax.numpy as jnp
from jax import lax
from jax.experimental import pallas as pl
from jax.experimental.pallas import tpu as pltpu

CHUNK = 16
NSLOT = 3


def kernel(x, W):
    logits = jnp.dot(x, W, preferred_element_type=jnp.float32)
    m_rows, n_half = logits.shape
    n_total = 2 * n_half
    half_rows = m_rows // 2
    n_fc = half_rows // CHUNK
    n_blocks = 2 * n_fc

    def body(logits_hbm, out_hbm, recv_vmem, lbuf, ostage,
             dsend, drecv, fsend, frecv, lsem, osem):
        i = pl.program_id(0)
        my_x = lax.axis_index("x")
        my_y = lax.axis_index("y")
        partner_y = (my_x, 1 - my_y)
        partner_x = (1 - my_x, my_y)

        def grow_of(step):
            ph = step // n_fc
            cc = step % n_fc
            owner = jnp.where(ph == 0, my_x, 1 - my_x)
            return owner * half_rows + cc * CHUNK

        def d_rdma(cc):
            base = my_x * half_rows + cc * CHUNK
            return pltpu.make_async_remote_copy(
                src_ref=logits_hbm.at[pl.ds(base, CHUNK)],
                dst_ref=recv_vmem.at[pl.ds(base, CHUNK)],
                send_sem=dsend.at[cc],
                recv_sem=drecv.at[cc],
                device_id=partner_y,
                device_id_type=pl.DeviceIdType.MESH,
            )

        def f_rdma(cc):
            base = my_x * half_rows + cc * CHUNK
            return pltpu.make_async_remote_copy(
                src_ref=recv_vmem.at[pl.ds(base, CHUNK)],
                dst_ref=recv_vmem.at[pl.ds(base, CHUNK)],
                send_sem=fsend.at[cc],
                recv_sem=frecv.at[cc],
                device_id=partner_x,
                device_id_type=pl.DeviceIdType.MESH,
            )

        def lcopy(step, slot):
            return pltpu.make_async_copy(
                logits_hbm.at[pl.ds(grow_of(step), CHUNK)],
                lbuf.at[slot],
                lsem.at[slot],
            )

        def ocopy(step):
            return pltpu.make_async_copy(
                ostage.at[step % NSLOT],
                out_hbm.at[pl.ds(grow_of(step), CHUNK)],
                osem.at[step % NSLOT],
            )

        @pl.when(i == 0)
        def _():
            barrier = pltpu.get_barrier_semaphore()
            for nbr in (partner_y, partner_x):
                pl.semaphore_signal(
                    barrier, inc=1, device_id=nbr,
                    device_id_type=pl.DeviceIdType.MESH,
                )
            pl.semaphore_wait(barrier, 2)
            for cc in range(n_fc):
                d_rdma(cc).start()
            for b in range(NSLOT):
                lcopy(b, b).start()

        @pl.when(jnp.logical_and(i >= 1, i + NSLOT - 1 < n_blocks))
        def _():
            lcopy(i + NSLOT - 1, (i + NSLOT - 1) % NSLOT).start()

        @pl.when(i < n_fc)
        def _():
            d_rdma(i).wait_recv()
            f_rdma(i).start()

        @pl.when(i >= n_fc)
        def _():
            f_rdma(i - n_fc).wait_recv()
            d_rdma(i - n_fc).wait_send()
            f_rdma(i - n_fc).wait_send()

        @pl.when(i >= NSLOT)
        def _():
            ocopy(i - NSLOT).wait()

        lcopy(i, i % NSLOT).wait()
        v_l = lbuf[i % NSLOT]
        v_r = recv_vmem[pl.ds(grow_of(i), CHUNK), :]
        m = jnp.maximum(
            jnp.max(v_l, axis=-1, keepdims=True),
            jnp.max(v_r, axis=-1, keepdims=True),
        )
        e_l = jnp.exp(v_l - m)
        e_r = jnp.exp(v_r - m)
        r = 1.0 / (
            jnp.sum(e_l, axis=-1, keepdims=True)
            + jnp.sum(e_r, axis=-1, keepdims=True)
        )
        slot = i % NSLOT
        ostage[slot, :, pl.ds(my_y * n_half, n_half)] = e_l * r
        ostage[slot, :, pl.ds((1 - my_y) * n_half, n_half)] = e_r * r
        ocopy(i).start()

        @pl.when(i == n_blocks - 1)
        def _():
            for d in (NSLOT - 1, NSLOT - 2, 0):
                ocopy(i - d).wait()

    return pl.pallas_call(
        body,
        grid=(n_blocks,),
        out_shape=jax.ShapeDtypeStruct((m_rows, n_total), jnp.float32),
        in_specs=[
            pl.BlockSpec(memory_space=pl.ANY),
        ],
        out_specs=pl.BlockSpec(memory_space=pl.ANY),
        scratch_shapes=[
            pltpu.VMEM((m_rows, n_half), jnp.float32),
            pltpu.VMEM((NSLOT, CHUNK, n_half), jnp.float32),
            pltpu.VMEM((NSLOT, CHUNK, n_total), jnp.float32),
            pltpu.SemaphoreType.DMA((n_fc,)),
            pltpu.SemaphoreType.DMA((n_fc,)),
            pltpu.SemaphoreType.DMA((n_fc,)),
            pltpu.SemaphoreType.DMA((n_fc,)),
            pltpu.SemaphoreType.DMA((NSLOT,)),
            pltpu.SemaphoreType.DMA((NSLOT,)),
        ],
        compiler_params=pltpu.CompilerParams(
            collective_id=0,
            dimension_semantics=("arbitrary",),
            vmem_limit_bytes=60 * 1024 * 1024,
        ),
    )(logits)
